# baseline (speedup 1.0000x reference)
import functools

import jax
import jax.numpy as jnp
from jax import lax
from jax.experimental import pallas as pl
from jax.experimental.pallas import tpu as pltpu
from jax.experimental.pallas import tpu_sc as plsc

_EPS = 1e-08
_N = 3276800
_NC = 2
_NS = 16
_NW = _NC * _NS
_PER_W = _N // _NW
_CHUNK = 12800
_NCHUNK = _PER_W // _CHUNK
_L = 16


def _body(x_hbm, g_hbm, mins_hbm, maxs_hbm, out_hbm,
          scale_v, offs_v, xb0, gb0, ob0, xb1, gb1, ob1,
          sem_ld0, sem_ld1, sem_st0, sem_st1):
    wid = lax.axis_index("s") * _NC + lax.axis_index("c")
    base = wid * _PER_W

    xbufs = (xb0, xb1)
    gbufs = (gb0, gb1)
    obufs = (ob0, ob1)
    ld_sems = (sem_ld0, sem_ld1)
    st_sems = (sem_st0, sem_st1)

    def start_load(c):
        b = c % 2
        off = base + c * _CHUNK
        hx = pltpu.async_copy(x_hbm.at[pl.ds(off, _CHUNK)], xbufs[b], ld_sems[b])
        hg = pltpu.async_copy(g_hbm.at[pl.ds(off, _CHUNK)], gbufs[b], ld_sems[b])
        return (hx, hg)

    h_ld = [None] * _NCHUNK
    h_ld[0] = start_load(0)

    pltpu.sync_copy(mins_hbm, scale_v)
    pltpu.sync_copy(maxs_hbm, offs_v)
    m = scale_v[...]
    M = offs_v[...]
    sc = (1.0 - 2.0 * _EPS) / (M - m)
    scale_v[...] = sc
    offs_v[...] = _EPS - m * sc

    h_st = [None] * _NCHUNK
    for c in range(_NCHUNK):
        b = c % 2
        if c + 1 < _NCHUNK:
            h_ld[c + 1] = start_load(c + 1)
        h_ld[c][0].wait()
        h_ld[c][1].wait()
        if c >= 2:
            h_st[c - 2].wait()

        xb, gb, ob = xbufs[b], gbufs[b], obufs[b]

        @pl.loop(0, _CHUNK // _L, unroll=8)
        def vec_body(i):
            s = pl.ds(i * _L, _L)
            idx = gb[s] - 1
            sg = plsc.load_gather(scale_v, [idx])
            og = plsc.load_gather(offs_v, [idx])
            ob[s] = xb[s] * sg + og

        off = base + c * _CHUNK
        h_st[c] = pltpu.async_copy(ob, out_hbm.at[pl.ds(off, _CHUNK)], st_sems[b])

    h_st[_NCHUNK - 2].wait()
    h_st[_NCHUNK - 1].wait()


@jax.jit
def _run(x, group, mins, maxs):
    mesh = plsc.VectorSubcoreMesh(core_axis_name="c", subcore_axis_name="s")
    kern = functools.partial(
        pl.kernel,
        mesh=mesh,
        compiler_params=pltpu.CompilerParams(needs_layout_passes=False),
        out_type=jax.ShapeDtypeStruct((_N,), jnp.float32),
        scratch_types=[
            pltpu.VMEM((_L,), jnp.float32),
            pltpu.VMEM((_L,), jnp.float32),
            pltpu.VMEM((_CHUNK,), jnp.float32),
            pltpu.VMEM((_CHUNK,), jnp.int32),
            pltpu.VMEM((_CHUNK,), jnp.float32),
            pltpu.VMEM((_CHUNK,), jnp.float32),
            pltpu.VMEM((_CHUNK,), jnp.int32),
            pltpu.VMEM((_CHUNK,), jnp.float32),
            pltpu.SemaphoreType.DMA,
            pltpu.SemaphoreType.DMA,
            pltpu.SemaphoreType.DMA,
            pltpu.SemaphoreType.DMA,
        ],
    )(_body)
    return kern(x, group, mins, maxs)


def kernel(x, group, mins, maxs):
    return _run(x, group, mins, maxs)


# speedup vs baseline: 2.4573x; 2.4573x over previous
import functools

import jax
import jax.numpy as jnp
from jax import lax
from jax.experimental import pallas as pl
from jax.experimental.pallas import tpu as pltpu
from jax.experimental.pallas import tpu_sc as plsc

_EPS = 1e-08
_N = 3276800
_NC = 2
_NS = 16
_NW = _NC * _NS
_PER_W = _N // _NW
_CHUNK = 12800
_NCHUNK = _PER_W // _CHUNK
_L = 16


def _body(x_hbm, g_hbm, mins_hbm, maxs_hbm, out_hbm,
          scale_v, offs_v, xb0, gb0, ob0, xb1, gb1, ob1,
          sem_ld0, sem_ld1, sem_st0, sem_st1):
    wid = lax.axis_index("s") * _NC + lax.axis_index("c")
    base = wid * _PER_W

    xbufs = (xb0, xb1)
    gbufs = (gb0, gb1)
    obufs = (ob0, ob1)
    ld_sems = (sem_ld0, sem_ld1)
    st_sems = (sem_st0, sem_st1)

    def start_load(c):
        b = c % 2
        off = base + c * _CHUNK
        hx = pltpu.async_copy(x_hbm.at[pl.ds(off, _CHUNK)], xbufs[b], ld_sems[b])
        hg = pltpu.async_copy(g_hbm.at[pl.ds(off, _CHUNK)], gbufs[b], ld_sems[b])
        return (hx, hg)

    h_ld = [None] * _NCHUNK
    h_ld[0] = start_load(0)

    pltpu.sync_copy(mins_hbm, scale_v)
    pltpu.sync_copy(maxs_hbm, offs_v)
    m = scale_v[...]
    M = offs_v[...]
    sc = (1.0 - 2.0 * _EPS) / (M - m)
    scale_v[...] = sc
    offs_v[...] = _EPS - m * sc

    h_st = [None] * _NCHUNK
    for c in range(_NCHUNK):
        b = c % 2
        if c + 1 < _NCHUNK:
            h_ld[c + 1] = start_load(c + 1)
        h_ld[c][0].wait()
        h_ld[c][1].wait()
        if c >= 2:
            h_st[c - 2].wait()

        xb, gb, ob = xbufs[b], gbufs[b], obufs[b]

        @plsc.parallel_loop(0, _CHUNK // _L, unroll=8)
        def vec_body(i):
            s = pl.ds(i * _L, _L)
            idx = gb[s] - 1
            sg = plsc.load_gather(scale_v, [idx])
            og = plsc.load_gather(offs_v, [idx])
            ob[s] = xb[s] * sg + og

        off = base + c * _CHUNK
        h_st[c] = pltpu.async_copy(ob, out_hbm.at[pl.ds(off, _CHUNK)], st_sems[b])

    h_st[_NCHUNK - 2].wait()
    h_st[_NCHUNK - 1].wait()


@jax.jit
def _run(x, group, mins, maxs):
    mesh = plsc.VectorSubcoreMesh(core_axis_name="c", subcore_axis_name="s")
    kern = functools.partial(
        pl.kernel,
        mesh=mesh,
        compiler_params=pltpu.CompilerParams(needs_layout_passes=False),
        out_type=jax.ShapeDtypeStruct((_N,), jnp.float32),
        scratch_types=[
            pltpu.VMEM((_L,), jnp.float32),
            pltpu.VMEM((_L,), jnp.float32),
            pltpu.VMEM((_CHUNK,), jnp.float32),
            pltpu.VMEM((_CHUNK,), jnp.int32),
            pltpu.VMEM((_CHUNK,), jnp.float32),
            pltpu.VMEM((_CHUNK,), jnp.float32),
            pltpu.VMEM((_CHUNK,), jnp.int32),
            pltpu.VMEM((_CHUNK,), jnp.float32),
            pltpu.SemaphoreType.DMA,
            pltpu.SemaphoreType.DMA,
            pltpu.SemaphoreType.DMA,
            pltpu.SemaphoreType.DMA,
        ],
    )(_body)
    return kern(x, group, mins, maxs)


def kernel(x, group, mins, maxs):
    return _run(x, group, mins, maxs)


# speedup vs baseline: 2.4619x; 1.0019x over previous
import functools

import jax
import jax.numpy as jnp
from jax import lax
from jax.experimental import pallas as pl
from jax.experimental.pallas import tpu as pltpu
from jax.experimental.pallas import tpu_sc as plsc

_EPS = 1e-08
_N = 3276800
_NC = 2
_NS = 16
_NW = _NC * _NS
_PER_W = _N // _NW
_CHUNK = 12800
_NCHUNK = _PER_W // _CHUNK
_L = 16


def _body(x_hbm, g_hbm, mins_hbm, maxs_hbm, out_hbm,
          scale_v, offs_v, xb0, gb0, ob0, xb1, gb1, ob1,
          sem_ld0, sem_ld1, sem_st0, sem_st1):
    wid = lax.axis_index("s") * _NC + lax.axis_index("c")
    base = wid * _PER_W

    xbufs = (xb0, xb1)
    gbufs = (gb0, gb1)
    obufs = (ob0, ob1)
    ld_sems = (sem_ld0, sem_ld1)
    st_sems = (sem_st0, sem_st1)

    def start_load(c):
        b = c % 2
        off = base + c * _CHUNK
        hx = pltpu.async_copy(x_hbm.at[pl.ds(off, _CHUNK)], xbufs[b], ld_sems[b])
        hg = pltpu.async_copy(g_hbm.at[pl.ds(off, _CHUNK)], gbufs[b], ld_sems[b])
        return (hx, hg)

    h_ld = [None] * _NCHUNK
    h_ld[0] = start_load(0)

    pltpu.sync_copy(mins_hbm, scale_v)
    pltpu.sync_copy(maxs_hbm, offs_v)
    m = scale_v[...]
    M = offs_v[...]
    sc = (1.0 - 2.0 * _EPS) / (M - m)
    scale_v[...] = sc
    offs_v[...] = _EPS - m * sc

    h_st = [None] * _NCHUNK
    for c in range(_NCHUNK):
        b = c % 2
        if c + 1 < _NCHUNK:
            h_ld[c + 1] = start_load(c + 1)
        h_ld[c][0].wait()
        h_ld[c][1].wait()
        if c >= 2:
            h_st[c - 2].wait()

        xb, gb, ob = xbufs[b], gbufs[b], obufs[b]

        @plsc.parallel_loop(0, _CHUNK // _L, unroll=8)
        def vec_body(i):
            s = pl.ds(i * _L, _L)
            idx = gb[s] - 1
            sg = plsc.load_gather(scale_v, [idx])
            og = plsc.load_gather(offs_v, [idx])
            ob[s] = xb[s] * sg + og

        off = base + c * _CHUNK
        h_st[c] = pltpu.async_copy(ob, out_hbm.at[pl.ds(off, _CHUNK)], st_sems[b])

    h_st[_NCHUNK - 2].wait()
    h_st[_NCHUNK - 1].wait()


@jax.jit
def _run(x, group, mins, maxs):
    mesh = plsc.VectorSubcoreMesh(core_axis_name="c", subcore_axis_name="s")
    kern = functools.partial(
        pl.kernel,
        mesh=mesh,
        compiler_params=pltpu.CompilerParams(
            needs_layout_passes=False, skip_device_barrier=True),
        out_type=jax.ShapeDtypeStruct((_N,), jnp.float32),
        scratch_types=[
            pltpu.VMEM((_L,), jnp.float32),
            pltpu.VMEM((_L,), jnp.float32),
            pltpu.VMEM((_CHUNK,), jnp.float32),
            pltpu.VMEM((_CHUNK,), jnp.int32),
            pltpu.VMEM((_CHUNK,), jnp.float32),
            pltpu.VMEM((_CHUNK,), jnp.float32),
            pltpu.VMEM((_CHUNK,), jnp.int32),
            pltpu.VMEM((_CHUNK,), jnp.float32),
            pltpu.SemaphoreType.DMA,
            pltpu.SemaphoreType.DMA,
            pltpu.SemaphoreType.DMA,
            pltpu.SemaphoreType.DMA,
        ],
    )(_body)
    return kern(x, group, mins, maxs)


def kernel(x, group, mins, maxs):
    return _run(x, group, mins, maxs)


# speedup vs baseline: 2.5396x; 1.0316x over previous
import functools

import jax
import jax.numpy as jnp
from jax import lax
from jax.experimental import pallas as pl
from jax.experimental.pallas import tpu as pltpu
from jax.experimental.pallas import tpu_sc as plsc

_EPS = 1e-08
_N = 3276800
_NC = 2
_NS = 16
_NW = _NC * _NS
_PER_W = _N // _NW
_CHUNK = 12800
_NCHUNK = _PER_W // _CHUNK
_L = 16


def _body(x_hbm, g_hbm, mins_hbm, maxs_hbm, out_hbm,
          scale_v, offs_v, xb0, gb0, ob0, xb1, gb1, ob1,
          sem_ld0, sem_ld1, sem_st0, sem_st1):
    wid = lax.axis_index("s") * _NC + lax.axis_index("c")
    base = wid * _PER_W

    xbufs = (xb0, xb1)
    gbufs = (gb0, gb1)
    obufs = (ob0, ob1)
    ld_sems = (sem_ld0, sem_ld1)
    st_sems = (sem_st0, sem_st1)

    def start_load(c, b):
        off = base + c * _CHUNK
        pltpu.async_copy(x_hbm.at[pl.ds(off, _CHUNK)], xbufs[b], ld_sems[b])
        pltpu.async_copy(g_hbm.at[pl.ds(off, _CHUNK)], gbufs[b], ld_sems[b])

    def wait_load(b):
        pltpu.make_async_copy(
            x_hbm.at[pl.ds(0, _CHUNK)], xbufs[b], ld_sems[b]).wait()
        pltpu.make_async_copy(
            g_hbm.at[pl.ds(0, _CHUNK)], gbufs[b], ld_sems[b]).wait()

    def wait_store(b):
        pltpu.make_async_copy(
            obufs[b], out_hbm.at[pl.ds(0, _CHUNK)], st_sems[b]).wait()

    def compute(b):
        xb, gb, ob = xbufs[b], gbufs[b], obufs[b]

        @plsc.parallel_loop(0, _CHUNK // _L, unroll=8)
        def vec_body(i):
            s = pl.ds(i * _L, _L)
            idx = gb[s] - 1
            sg = plsc.load_gather(scale_v, [idx])
            og = plsc.load_gather(offs_v, [idx])
            ob[s] = xb[s] * sg + og

    def start_store(c, b):
        off = base + c * _CHUNK
        pltpu.async_copy(obufs[b], out_hbm.at[pl.ds(off, _CHUNK)], st_sems[b])

    start_load(0, 0)

    pltpu.sync_copy(mins_hbm, scale_v)
    pltpu.sync_copy(maxs_hbm, offs_v)
    m = scale_v[...]
    M = offs_v[...]
    sc = (1.0 - 2.0 * _EPS) / (M - m)
    scale_v[...] = sc
    offs_v[...] = _EPS - m * sc

    @pl.loop(0, _NCHUNK, step=2)
    def chunk_pair(c):
        start_load(c + 1, 1)
        wait_load(0)

        @pl.when(c >= 2)
        def _():
            wait_store(0)

        compute(0)
        start_store(c, 0)

        @pl.when(c + 2 < _NCHUNK)
        def _():
            start_load(c + 2, 0)

        wait_load(1)

        @pl.when(c >= 2)
        def _():
            wait_store(1)

        compute(1)
        start_store(c + 1, 1)

    wait_store(0)
    wait_store(1)


@jax.jit
def _run(x, group, mins, maxs):
    mesh = plsc.VectorSubcoreMesh(core_axis_name="c", subcore_axis_name="s")
    kern = functools.partial(
        pl.kernel,
        mesh=mesh,
        compiler_params=pltpu.CompilerParams(needs_layout_passes=False),
        out_type=jax.ShapeDtypeStruct((_N,), jnp.float32),
        scratch_types=[
            pltpu.VMEM((_L,), jnp.float32),
            pltpu.VMEM((_L,), jnp.float32),
            pltpu.VMEM((_CHUNK,), jnp.float32),
            pltpu.VMEM((_CHUNK,), jnp.int32),
            pltpu.VMEM((_CHUNK,), jnp.float32),
            pltpu.VMEM((_CHUNK,), jnp.float32),
            pltpu.VMEM((_CHUNK,), jnp.int32),
            pltpu.VMEM((_CHUNK,), jnp.float32),
            pltpu.SemaphoreType.DMA,
            pltpu.SemaphoreType.DMA,
            pltpu.SemaphoreType.DMA,
            pltpu.SemaphoreType.DMA,
        ],
    )(_body)
    return kern(x, group, mins, maxs)


def kernel(x, group, mins, maxs):
    return _run(x, group, mins, maxs)


# speedup vs baseline: 2.6741x; 1.0530x over previous
import functools

import jax
import jax.numpy as jnp
from jax import lax
from jax.experimental import pallas as pl
from jax.experimental.pallas import tpu as pltpu
from jax.experimental.pallas import tpu_sc as plsc

_EPS = 1e-08
_N = 3276800
_NC = 2
_NS = 16
_NW = _NC * _NS
_PER_W = _N // _NW
_CHUNK = 12800
_NCHUNK = _PER_W // _CHUNK
_L = 16


def _body(x_hbm, g_hbm, mins_hbm, maxs_hbm, out_hbm,
          scale_v, offs_v, xb0, gb0, ob0, xb1, gb1, ob1,
          sem_ld0, sem_ld1, sem_st0, sem_st1):
    wid = lax.axis_index("s") * _NC + lax.axis_index("c")
    base = wid * _PER_W

    xbufs = (xb0, xb1)
    gbufs = (gb0, gb1)
    obufs = (ob0, ob1)
    ld_sems = (sem_ld0, sem_ld1)
    st_sems = (sem_st0, sem_st1)

    def start_load(c, b):
        off = base + c * _CHUNK
        pltpu.async_copy(x_hbm.at[pl.ds(off, _CHUNK)], xbufs[b], ld_sems[b])
        pltpu.async_copy(g_hbm.at[pl.ds(off, _CHUNK)], gbufs[b], ld_sems[b])

    def wait_load(b):
        pltpu.make_async_copy(
            x_hbm.at[pl.ds(0, _CHUNK)], xbufs[b], ld_sems[b]).wait()
        pltpu.make_async_copy(
            g_hbm.at[pl.ds(0, _CHUNK)], gbufs[b], ld_sems[b]).wait()

    def wait_store(b):
        pltpu.make_async_copy(
            obufs[b], out_hbm.at[pl.ds(0, _CHUNK)], st_sems[b]).wait()

    def compute(b, scale_reg, offs_reg):
        xb, gb, ob = xbufs[b], gbufs[b], obufs[b]

        @plsc.parallel_loop(0, _CHUNK // _L, unroll=8)
        def vec_body(i):
            s = pl.ds(i * _L, _L)
            idx = gb[s] - 1
            sg = jnp.take_along_axis(
                scale_reg, idx, axis=0, mode="promise_in_bounds")
            og = jnp.take_along_axis(
                offs_reg, idx, axis=0, mode="promise_in_bounds")
            ob[s] = xb[s] * sg + og

    def start_store(c, b):
        off = base + c * _CHUNK
        pltpu.async_copy(obufs[b], out_hbm.at[pl.ds(off, _CHUNK)], st_sems[b])

    start_load(0, 0)

    pltpu.sync_copy(mins_hbm, scale_v)
    pltpu.sync_copy(maxs_hbm, offs_v)
    m = scale_v[...]
    M = offs_v[...]
    scale_reg = (1.0 - 2.0 * _EPS) / (M - m)
    offs_reg = _EPS - m * scale_reg

    @pl.loop(0, _NCHUNK, step=2)
    def chunk_pair(c):
        start_load(c + 1, 1)
        wait_load(0)

        @pl.when(c >= 2)
        def _():
            wait_store(0)

        compute(0, scale_reg, offs_reg)
        start_store(c, 0)

        @pl.when(c + 2 < _NCHUNK)
        def _():
            start_load(c + 2, 0)

        wait_load(1)

        @pl.when(c >= 2)
        def _():
            wait_store(1)

        compute(1, scale_reg, offs_reg)
        start_store(c + 1, 1)

    wait_store(0)
    wait_store(1)


@jax.jit
def _run(x, group, mins, maxs):
    mesh = plsc.VectorSubcoreMesh(core_axis_name="c", subcore_axis_name="s")
    kern = functools.partial(
        pl.kernel,
        mesh=mesh,
        compiler_params=pltpu.CompilerParams(needs_layout_passes=False),
        out_type=jax.ShapeDtypeStruct((_N,), jnp.float32),
        scratch_types=[
            pltpu.VMEM((_L,), jnp.float32),
            pltpu.VMEM((_L,), jnp.float32),
            pltpu.VMEM((_CHUNK,), jnp.float32),
            pltpu.VMEM((_CHUNK,), jnp.int32),
            pltpu.VMEM((_CHUNK,), jnp.float32),
            pltpu.VMEM((_CHUNK,), jnp.float32),
            pltpu.VMEM((_CHUNK,), jnp.int32),
            pltpu.VMEM((_CHUNK,), jnp.float32),
            pltpu.SemaphoreType.DMA,
            pltpu.SemaphoreType.DMA,
            pltpu.SemaphoreType.DMA,
            pltpu.SemaphoreType.DMA,
        ],
    )(_body)
    return kern(x, group, mins, maxs)


def kernel(x, group, mins, maxs):
    return _run(x, group, mins, maxs)


# speedup vs baseline: 2.7409x; 1.0250x over previous
import functools

import jax
import jax.numpy as jnp
from jax import lax
from jax.experimental import pallas as pl
from jax.experimental.pallas import tpu as pltpu
from jax.experimental.pallas import tpu_sc as plsc

_EPS = 1e-08
_N = 3276800
_NC = 2
_NS = 16
_NW = _NC * _NS
_PER_W = _N // _NW
_NBUF = 4
_CHUNK = 6400
_NCHUNK = _PER_W // _CHUNK
_L = 16


def _body(x_hbm, g_hbm, mins_hbm, maxs_hbm, out_hbm, scale_v, offs_v, *rest):
    xbufs = rest[0:_NBUF]
    gbufs = rest[_NBUF:2 * _NBUF]
    obufs = rest[2 * _NBUF:3 * _NBUF]
    ld_sems = rest[3 * _NBUF:4 * _NBUF]
    st_sems = rest[4 * _NBUF:5 * _NBUF]

    wid = lax.axis_index("s") * _NC + lax.axis_index("c")
    base = wid * _PER_W

    def start_load(c, b):
        off = base + c * _CHUNK
        pltpu.async_copy(x_hbm.at[pl.ds(off, _CHUNK)], xbufs[b], ld_sems[b])
        pltpu.async_copy(g_hbm.at[pl.ds(off, _CHUNK)], gbufs[b], ld_sems[b])

    def wait_load(b):
        pltpu.make_async_copy(
            x_hbm.at[pl.ds(0, _CHUNK)], xbufs[b], ld_sems[b]).wait()
        pltpu.make_async_copy(
            g_hbm.at[pl.ds(0, _CHUNK)], gbufs[b], ld_sems[b]).wait()

    def wait_store(b):
        pltpu.make_async_copy(
            obufs[b], out_hbm.at[pl.ds(0, _CHUNK)], st_sems[b]).wait()

    def start_store(c, b):
        off = base + c * _CHUNK
        pltpu.async_copy(obufs[b], out_hbm.at[pl.ds(off, _CHUNK)], st_sems[b])

    def compute(b, scale_reg, offs_reg):
        xb, gb, ob = xbufs[b], gbufs[b], obufs[b]

        @plsc.parallel_loop(0, _CHUNK // _L, unroll=8)
        def vec_body(i):
            s = pl.ds(i * _L, _L)
            idx = gb[s] - 1
            sg = jnp.take_along_axis(
                scale_reg, idx, axis=0, mode="promise_in_bounds")
            og = jnp.take_along_axis(
                offs_reg, idx, axis=0, mode="promise_in_bounds")
            ob[s] = xb[s] * sg + og

    for b in range(_NBUF - 1):
        start_load(b, b)

    pltpu.sync_copy(mins_hbm, scale_v)
    pltpu.sync_copy(maxs_hbm, offs_v)
    m = scale_v[...]
    M = offs_v[...]
    scale_reg = (1.0 - 2.0 * _EPS) / (M - m)
    offs_reg = _EPS - m * scale_reg

    @pl.loop(0, _NCHUNK, step=_NBUF)
    def chunk_ring(c):
        for j in range(_NBUF):
            b = j
            cj = c + j

            wait_load(b)

            @pl.when(cj >= _NBUF)
            def _():
                wait_store(b)

            compute(b, scale_reg, offs_reg)
            start_store(cj, b)

            @pl.when(cj + _NBUF - 1 < _NCHUNK)
            def _():
                start_load(cj + _NBUF - 1, (b + _NBUF - 1) % _NBUF)

    for b in range(_NBUF):
        wait_store(b)


@jax.jit
def _run(x, group, mins, maxs):
    mesh = plsc.VectorSubcoreMesh(core_axis_name="c", subcore_axis_name="s")
    kern = functools.partial(
        pl.kernel,
        mesh=mesh,
        compiler_params=pltpu.CompilerParams(needs_layout_passes=False),
        out_type=jax.ShapeDtypeStruct((_N,), jnp.float32),
        scratch_types=(
            [
                pltpu.VMEM((_L,), jnp.float32),
                pltpu.VMEM((_L,), jnp.float32),
            ]
            + [pltpu.VMEM((_CHUNK,), jnp.float32) for _ in range(_NBUF)]
            + [pltpu.VMEM((_CHUNK,), jnp.int32) for _ in range(_NBUF)]
            + [pltpu.VMEM((_CHUNK,), jnp.float32) for _ in range(_NBUF)]
            + [pltpu.SemaphoreType.DMA for _ in range(_NBUF)]
            + [pltpu.SemaphoreType.DMA for _ in range(_NBUF)]
        ),
    )(_body)
    return kern(x, group, mins, maxs)


def kernel(x, group, mins, maxs):
    return _run(x, group, mins, maxs)
